# baseline (speedup 1.0000x reference)
import functools

import jax
import jax.numpy as jnp
from jax import lax
from jax.experimental import pallas as pl
from jax.experimental.pallas import tpu as pltpu
from jax.experimental.pallas import tpu_sc as plsc

VOCAB = 1000
EMB = 16
B, L = 4096, 20
N = B * L

_NC, _NS = 2, 16
_NW = _NC * _NS
_B_PER_W = B // _NW
_LH = L // 2

_SC_MESH = plsc.VectorSubcoreMesh(core_axis_name="c", subcore_axis_name="s")


def _make_sc_gather(l0):
    @functools.partial(
        pl.kernel,
        out_type=jax.ShapeDtypeStruct((_LH, EMB, B), jnp.float32),
        mesh=_SC_MESH,
        scratch_types=[
            pltpu.VMEM((_LH, _B_PER_W), jnp.int32),
            pltpu.VMEM((VOCAB, EMB), jnp.float32),
            pltpu.VMEM((_LH, EMB, _B_PER_W), jnp.float32),
            pltpu.SemaphoreType.DMA,
        ],
        compiler_params=pltpu.CompilerParams(
            use_tc_tiling_on_sc=False,
            needs_layout_passes=False,
            disable_bounds_checks=True,
        ),
        name=f"sc_gather_emb_l{l0}",
    )
    def gather(table_hbm, xt_hbm, emb_hbm, idx_v, tab_v, trans_v, wsem):
        wid = lax.axis_index("s") * _NC + lax.axis_index("c")
        b0 = wid * _B_PER_W
        pltpu.sync_copy(
            xt_hbm.at[pl.ds(l0, _LH), pl.ds(b0, _B_PER_W)], idx_v
        )
        pltpu.sync_copy(table_hbm, tab_v)

        def body(l, _):
            for k in range(_B_PER_W // 16):
                idxv = idx_v[l, pl.ds(k * 16, 16)]
                for e in range(EMB):
                    col = jnp.full((16,), e, jnp.int32)
                    vals = plsc.load_gather(tab_v, [idxv, col])
                    trans_v[l, e, pl.ds(k * 16, 16)] = vals
            return 0

        lax.fori_loop(0, _LH, body, 0)
        pltpu.async_copy(
            trans_v, emb_hbm.at[:, :, pl.ds(b0, _B_PER_W)], wsem
        ).wait()

    return gather


_sc_gather_a = _make_sc_gather(0)
_sc_gather_b = _make_sc_gather(_LH)

_BT = 2048


def _proj_body(w_ref, b_ref, emb_ref, out_ref):
    e = emb_ref[0]
    m = lax.dot_general(
        w_ref[...], e, (((0,), (0,)), ((), ())),
        preferred_element_type=jnp.float32,
    )
    out_ref[0] = m + b_ref[...]


def _proj_body_alias(w_ref, b_ref, emb_ref, prev_ref, out_ref):
    del prev_ref
    _proj_body(w_ref, b_ref, emb_ref, out_ref)


_W_SPEC = pl.BlockSpec((EMB, VOCAB), lambda l, j: (0, 0))
_B_SPEC = pl.BlockSpec((VOCAB, 1), lambda l, j: (0, 0))
_EMB_SPEC = pl.BlockSpec((1, EMB, _BT), lambda l, j: (l, 0, j))
_GRID = (_LH, B // _BT)
_OUT_SHAPE = jax.ShapeDtypeStruct((L, VOCAB, B), jnp.float32)


def _tc_project_first(W, b2, emb):
    return pl.pallas_call(
        _proj_body,
        grid=_GRID,
        in_specs=[_W_SPEC, _B_SPEC, _EMB_SPEC],
        out_specs=pl.BlockSpec((1, VOCAB, _BT), lambda l, j: (l, 0, j)),
        out_shape=_OUT_SHAPE,
    )(W, b2, emb)


def _tc_project_second(W, b2, emb, prev):
    return pl.pallas_call(
        _proj_body_alias,
        grid=_GRID,
        in_specs=[
            _W_SPEC,
            _B_SPEC,
            _EMB_SPEC,
            pl.BlockSpec(memory_space=pltpu.MemorySpace.HBM),
        ],
        out_specs=pl.BlockSpec((1, VOCAB, _BT), lambda l, j: (l + _LH, 0, j)),
        out_shape=_OUT_SHAPE,
        input_output_aliases={3: 0},
    )(W, b2, emb, prev)


def kernel(x, table, W, b):
    xt = x.astype(jnp.int32).T
    b2 = b.reshape(VOCAB, 1)
    emb_a = _sc_gather_a(table, xt)
    emb_b = _sc_gather_b(table, xt)
    out1 = _tc_project_first(W, b2, emb_a)
    outT = _tc_project_second(W, b2, emb_b, out1)
    return jnp.transpose(outT, (2, 0, 1))


# speedup vs baseline: 1.0675x; 1.0675x over previous
import functools

import jax
import jax.numpy as jnp
from jax import lax
from jax.experimental import pallas as pl
from jax.experimental.pallas import tpu as pltpu
from jax.experimental.pallas import tpu_sc as plsc

VOCAB = 1000
EMB = 16
B, L = 4096, 20
N = B * L

_NC, _NS = 2, 16
_NW = _NC * _NS
_B_PER_W = B // _NW

_SC_MESH = plsc.VectorSubcoreMesh(core_axis_name="c", subcore_axis_name="s")


@functools.partial(
    pl.kernel,
    out_type=jax.ShapeDtypeStruct((L, EMB, B), jnp.float32),
    mesh=_SC_MESH,
    scratch_types=[
        pltpu.VMEM((L, _B_PER_W), jnp.int32),
        pltpu.VMEM((VOCAB, EMB), jnp.float32),
        pltpu.VMEM((L, EMB, _B_PER_W), jnp.float32),
        pltpu.SemaphoreType.DMA,
    ],
    compiler_params=pltpu.CompilerParams(
        use_tc_tiling_on_sc=False,
        needs_layout_passes=False,
        disable_bounds_checks=True,
    ),
)
def _sc_gather_emb(table_hbm, xt_hbm, emb_hbm, idx_v, tab_v, trans_v, wsem):
    wid = lax.axis_index("s") * _NC + lax.axis_index("c")
    b0 = wid * _B_PER_W
    pltpu.sync_copy(xt_hbm.at[:, pl.ds(b0, _B_PER_W)], idx_v)
    pltpu.sync_copy(table_hbm, tab_v)

    def body(l, _):
        for k in range(_B_PER_W // 16):
            idxv = idx_v[l, pl.ds(k * 16, 16)]
            for e in range(EMB):
                col = jnp.full((16,), e, jnp.int32)
                vals = plsc.load_gather(tab_v, [idxv, col])
                trans_v[l, e, pl.ds(k * 16, 16)] = vals
        return 0

    lax.fori_loop(0, L, body, 0)
    pltpu.async_copy(
        trans_v, emb_hbm.at[:, :, pl.ds(b0, _B_PER_W)], wsem
    ).wait()


_BT = 2048


def _proj_body(w_ref, b_ref, emb_ref, out_ref):
    e = emb_ref[0]
    m = lax.dot_general(
        w_ref[...], e, (((0,), (0,)), ((), ())),
        preferred_element_type=jnp.float32,
    )
    out_ref[0] = m + b_ref[...]


def _tc_project(W, b2, emb3):
    grid = (L, B // _BT)
    return pl.pallas_call(
        _proj_body,
        grid=grid,
        in_specs=[
            pl.BlockSpec((EMB, VOCAB), lambda l, j: (0, 0)),
            pl.BlockSpec((VOCAB, 1), lambda l, j: (0, 0)),
            pl.BlockSpec((1, EMB, _BT), lambda l, j: (l, 0, j)),
        ],
        out_specs=pl.BlockSpec((1, VOCAB, _BT), lambda l, j: (l, 0, j)),
        out_shape=jax.ShapeDtypeStruct((L, VOCAB, B), jnp.float32),
    )(W, b2, emb3)


def kernel(x, table, W, b):
    xt = x.astype(jnp.int32).T
    emb3 = _sc_gather_emb(table, xt)
    outT = _tc_project(W, b.reshape(VOCAB, 1), emb3)
    return jnp.transpose(outT, (2, 0, 1))


# speedup vs baseline: 1.1042x; 1.0343x over previous
import functools

import jax
import jax.numpy as jnp
from jax import lax
from jax.experimental import pallas as pl
from jax.experimental.pallas import tpu as pltpu
from jax.experimental.pallas import tpu_sc as plsc

VOCAB = 1000
EMB = 16
B, L = 4096, 20
N = B * L

_NC, _NS = 2, 16
_NW = _NC * _NS
_B_PER_W = B // _NW

_SC_MESH = plsc.VectorSubcoreMesh(core_axis_name="c", subcore_axis_name="s")


@functools.partial(
    pl.kernel,
    out_type=jax.ShapeDtypeStruct((L, EMB // 8, _NW, 8, _B_PER_W), jnp.float32),
    mesh=_SC_MESH,
    scratch_types=[
        pltpu.VMEM((L, _B_PER_W), jnp.int32),
        pltpu.VMEM((VOCAB, EMB), jnp.float32),
        pltpu.VMEM((L, EMB // 8, 8, _B_PER_W), jnp.float32),
        pltpu.SemaphoreType.DMA,
    ],
    compiler_params=pltpu.CompilerParams(
        use_tc_tiling_on_sc=False,
        needs_layout_passes=False,
        disable_bounds_checks=True,
    ),
)
def _sc_gather_emb(table_hbm, xt_hbm, emb_hbm, idx_v, tab_v, trans_v, wsem):
    wid = lax.axis_index("s") * _NC + lax.axis_index("c")
    b0 = wid * _B_PER_W
    pltpu.sync_copy(xt_hbm.at[:, pl.ds(b0, _B_PER_W)], idx_v)
    pltpu.sync_copy(table_hbm, tab_v)

    def body(l, _):
        for k in range(_B_PER_W // 16):
            idxv = idx_v[l, pl.ds(k * 16, 16)]
            for e in range(EMB):
                col = jnp.full((16,), e, jnp.int32)
                vals = plsc.load_gather(tab_v, [idxv, col])
                trans_v[l, e // 8, e % 8, pl.ds(k * 16, 16)] = vals
        return 0

    lax.fori_loop(0, L, body, 0)
    pltpu.async_copy(
        trans_v, emb_hbm.at[:, :, wid], wsem
    ).wait()


_BT = 2048


def _proj_body(w_ref, b_ref, emb_ref, out_ref):
    e = emb_ref[0]
    m = lax.dot_general(
        w_ref[...], e, (((0,), (0,)), ((), ())),
        preferred_element_type=jnp.float32,
    )
    out_ref[0] = m + b_ref[...]


def _tc_project(W, b2, emb3):
    grid = (L, B // _BT)
    return pl.pallas_call(
        _proj_body,
        grid=grid,
        in_specs=[
            pl.BlockSpec((EMB, VOCAB), lambda l, j: (0, 0)),
            pl.BlockSpec((VOCAB, 1), lambda l, j: (0, 0)),
            pl.BlockSpec((1, EMB, _BT), lambda l, j: (l, 0, j)),
        ],
        out_specs=pl.BlockSpec((1, VOCAB, _BT), lambda l, j: (l, 0, j)),
        out_shape=jax.ShapeDtypeStruct((L, VOCAB, B), jnp.float32),
    )(W, b2, emb3)


def kernel(x, table, W, b):
    xt = x.astype(jnp.int32).T
    emb5 = _sc_gather_emb(table, xt)
    emb3 = jnp.transpose(emb5, (0, 1, 3, 2, 4)).reshape(L, EMB, B)
    outT = _tc_project(W, b.reshape(VOCAB, 1), emb3)
    return jnp.transpose(outT, (2, 0, 1))
